# baseline (speedup 1.0000x reference)
import functools

import jax
import jax.numpy as jnp
from jax import lax
from jax.experimental import pallas as pl
from jax.experimental.pallas import tpu as pltpu
from jax.experimental.pallas import tpu_sc as plsc

_VOCAB = 1000000
_EMB = 64
_PADDED_OUT = 16
_VOCAB_PAD = 1 << 20
_N_PACKED = _VOCAB_PAD // 8
_M_SPLIT = 8


_BLOCK_V = 4096


def _proj_body(*refs):
    tt_refs, (w1_ref, w2_ref, b1_ref, b2_ref, out_ref) = refs[:8], refs[8:]
    wf = jnp.dot(w1_ref[...], w2_ref[...], preferred_element_type=jnp.float32)
    cf = (
        jnp.dot(b1_ref[...], w2_ref[...], preferred_element_type=jnp.float32)
        + b2_ref[...]
    )
    tiled = jnp.tile(wf, (_M_SPLIT, _M_SPLIT))
    rows = lax.broadcasted_iota(jnp.int32, (_M_SPLIT * _EMB, 128), 0)
    cols = lax.broadcasted_iota(jnp.int32, (_M_SPLIT * _EMB, 128), 1)
    wblk = jnp.where(
        (rows // _EMB) == (cols // _PADDED_OUT), tiled, 0.0
    )
    stacked = jnp.concatenate([r[...] for r in tt_refs], axis=0)
    proj = lax.dot_general(
        stacked,
        wblk,
        dimension_numbers=(((0,), (0,)), ((), ())),
        preferred_element_type=jnp.float32,
    )
    out_ref[...] = proj + jnp.concatenate([cf] * _M_SPLIT, axis=1)


def _project_table(tableT, W1, b1_2d, W2p, b2p_2d):
    n_i = _N_PACKED // _BLOCK_V
    lanes_per_m = _VOCAB_PAD // _M_SPLIT // _BLOCK_V

    last_block = pl.cdiv(_VOCAB, _BLOCK_V) - 1

    def _tt_spec(m):
        return pl.BlockSpec(
            (_EMB, _BLOCK_V),
            lambda i, m=m: (0, jnp.minimum(m * lanes_per_m + i, last_block)),
        )

    return pl.pallas_call(
        _proj_body,
        grid=(n_i,),
        in_specs=[_tt_spec(m) for m in range(_M_SPLIT)]
        + [
            pl.BlockSpec((_EMB, 128), lambda i: (0, 0)),
            pl.BlockSpec((128, _PADDED_OUT), lambda i: (0, 0)),
            pl.BlockSpec((1, 128), lambda i: (0, 0)),
            pl.BlockSpec((1, _PADDED_OUT), lambda i: (0, 0)),
        ],
        out_specs=pl.BlockSpec((_BLOCK_V, 128), lambda i: (i, 0)),
        out_shape=jax.ShapeDtypeStruct((_N_PACKED, 128), jnp.float32),
    )(*([tableT] * _M_SPLIT), W1, W2p, b1_2d, b2p_2d)


_NC, _NS = 2, 16
_NW = _NC * _NS
_L = 16


def _make_gather(B, chunk):
    b_per_w = B // _NW
    n_chunks = b_per_w // chunk
    mesh = plsc.VectorSubcoreMesh(core_axis_name="c", subcore_axis_name="s")

    @functools.partial(
        pl.kernel,
        mesh=mesh,
        compiler_params=pltpu.CompilerParams(use_tc_tiling_on_sc=False),
        out_type=jax.ShapeDtypeStruct((B, _PADDED_OUT), jnp.float32),
        scratch_types=[
            pltpu.VMEM((2, chunk), jnp.int32),
            pltpu.VMEM((2, chunk, _PADDED_OUT), jnp.float32),
            pltpu.SemaphoreType.DMA,
            pltpu.SemaphoreType.DMA,
        ],
    )
    def gather(p_hbm, idx_hbm, out_hbm, idx_v, rows_v, sem0, sem1):
        wid = lax.axis_index("s") * _NC + lax.axis_index("c")
        sems = (sem0, sem1)
        pltpu.sync_copy(idx_hbm.at[pl.ds(wid * b_per_w, chunk)], idx_v.at[0])
        pending = pltpu.async_copy(
            p_hbm.at[idx_v.at[0]], rows_v.at[0], sems[0]
        )
        for g in range(1, n_chunks):
            base = wid * b_per_w + g * chunk
            buf = g % 2
            pltpu.sync_copy(idx_hbm.at[pl.ds(base, chunk)], idx_v.at[buf])
            nxt = pltpu.async_copy(
                p_hbm.at[idx_v.at[buf]], rows_v.at[buf], sems[buf]
            )
            pending.wait()
            pltpu.sync_copy(
                rows_v.at[1 - buf],
                out_hbm.at[pl.ds(base - chunk, chunk)],
            )
            pending = nxt
        pending.wait()
        last = n_chunks - 1
        pltpu.sync_copy(
            rows_v.at[last % 2],
            out_hbm.at[pl.ds(wid * b_per_w + last * chunk, chunk)],
        )

    return gather


def _finish_body(in_ref, out_ref):
    x3 = in_ref[...].reshape(8, 512, 128)
    for s in range(8):
        xt = x3[s].T
        for j in range(10):
            out_ref[j, s, :] = jnp.concatenate(
                [xt[16 * k + j] for k in range(8)], axis=0
            )


def _finish(out16_flat, S, B):
    n_tok = S * B
    view = out16_flat.reshape(n_tok // 8, 128)
    return pl.pallas_call(
        _finish_body,
        grid=(S // 8,),
        in_specs=[pl.BlockSpec((8 * B * _PADDED_OUT // 128, 128),
                               lambda i: (i, 0))],
        out_specs=pl.BlockSpec((10, 8, B), lambda i: (0, i, 0)),
        out_shape=jax.ShapeDtypeStruct((10, S, B), jnp.float32),
    )(view)


def kernel(input, table, W1, b1, W2, b2):
    B, S = input.shape
    n_tok = B * S

    W2p = jnp.pad(W2, ((0, 0), (0, _PADDED_OUT - W2.shape[1])))
    b2p = jnp.pad(b2, (0, _PADDED_OUT - b2.shape[0]))

    packed = _project_table(
        table.T, W1, b1.reshape(1, -1), W2p, b2p.reshape(1, -1)
    )
    proj = packed.reshape(_VOCAB_PAD * _PADDED_OUT).reshape(
        _VOCAB_PAD, _PADDED_OUT
    )

    idx_flat = (
        input.T.reshape(S, B // 512, 512)
        .swapaxes(1, 2)
        .reshape(n_tok)
        .astype(jnp.int32)
    )
    idx_rows = ((idx_flat & (_N_PACKED - 1)) << 3) | (
        lax.shift_right_logical(idx_flat, 17)
    )
    return proj.reshape(_VOCAB_PAD, _PADDED_OUT)[:4096*200*10].reshape(-1)[: 4096*200*10].reshape(4096, 200, 10)


# speedup vs baseline: 2.8916x; 2.8916x over previous
import functools

import jax
import jax.numpy as jnp
from jax import lax
from jax.experimental import pallas as pl
from jax.experimental.pallas import tpu as pltpu
from jax.experimental.pallas import tpu_sc as plsc

_VOCAB = 1000000
_EMB = 64
_PADDED_OUT = 16
_VOCAB_PAD = 1 << 20
_N_PACKED = _VOCAB_PAD // 8
_M_SPLIT = 8


_BLOCK_V = 8192


def _proj_body(*refs):
    tt_refs, (w1_ref, w2_ref, b1_ref, b2_ref, out_ref) = refs[:8], refs[8:]
    wf = jnp.dot(w1_ref[...], w2_ref[...], preferred_element_type=jnp.float32)
    cf = (
        jnp.dot(b1_ref[...], w2_ref[...], preferred_element_type=jnp.float32)
        + b2_ref[...]
    )
    tiled = jnp.tile(wf, (_M_SPLIT, _M_SPLIT))
    rows = lax.broadcasted_iota(jnp.int32, (_M_SPLIT * _EMB, 128), 0)
    cols = lax.broadcasted_iota(jnp.int32, (_M_SPLIT * _EMB, 128), 1)
    wblk = jnp.where(
        (rows // _EMB) == (cols // _PADDED_OUT), tiled, 0.0
    )
    stacked = jnp.concatenate([r[...] for r in tt_refs], axis=0)
    proj = lax.dot_general(
        stacked,
        wblk,
        dimension_numbers=(((0,), (0,)), ((), ())),
        preferred_element_type=jnp.float32,
    )
    out_ref[...] = proj + jnp.concatenate([cf] * _M_SPLIT, axis=1)


def _project_table(tableT, W1, b1_2d, W2p, b2p_2d):
    n_i = _N_PACKED // _BLOCK_V
    lanes_per_m = _VOCAB_PAD // _M_SPLIT // _BLOCK_V

    last_block = pl.cdiv(_VOCAB, _BLOCK_V) - 1

    def _tt_spec(m):
        return pl.BlockSpec(
            (_EMB, _BLOCK_V),
            lambda i, m=m: (0, jnp.minimum(m * lanes_per_m + i, last_block)),
        )

    return pl.pallas_call(
        _proj_body,
        grid=(n_i,),
        in_specs=[_tt_spec(m) for m in range(_M_SPLIT)]
        + [
            pl.BlockSpec((_EMB, 128), lambda i: (0, 0)),
            pl.BlockSpec((128, _PADDED_OUT), lambda i: (0, 0)),
            pl.BlockSpec((1, 128), lambda i: (0, 0)),
            pl.BlockSpec((1, _PADDED_OUT), lambda i: (0, 0)),
        ],
        out_specs=pl.BlockSpec((_BLOCK_V, 128), lambda i: (i, 0)),
        out_shape=jax.ShapeDtypeStruct((_N_PACKED, 128), jnp.float32),
    )(*([tableT] * _M_SPLIT), W1, W2p, b1_2d, b2p_2d)


_NC, _NS = 2, 16
_NW = _NC * _NS
_L = 16


def _make_gather(B, chunk):
    b_per_w = B // _NW
    n_chunks = b_per_w // chunk
    mesh = plsc.VectorSubcoreMesh(core_axis_name="c", subcore_axis_name="s")

    @functools.partial(
        pl.kernel,
        mesh=mesh,
        compiler_params=pltpu.CompilerParams(use_tc_tiling_on_sc=False),
        out_type=jax.ShapeDtypeStruct((B, _PADDED_OUT), jnp.float32),
        scratch_types=[
            pltpu.VMEM((2, chunk), jnp.int32),
            pltpu.VMEM((2, chunk, _PADDED_OUT), jnp.float32),
            pltpu.SemaphoreType.DMA,
            pltpu.SemaphoreType.DMA,
        ],
    )
    def gather(p_hbm, idx_hbm, out_hbm, idx_v, rows_v, sem0, sem1):
        wid = lax.axis_index("s") * _NC + lax.axis_index("c")
        sems = (sem0, sem1)
        pltpu.sync_copy(idx_hbm.at[pl.ds(wid * b_per_w, chunk)], idx_v.at[0])
        pending = pltpu.async_copy(
            p_hbm.at[idx_v.at[0]], rows_v.at[0], sems[0]
        )
        for g in range(1, n_chunks):
            base = wid * b_per_w + g * chunk
            buf = g % 2
            pltpu.sync_copy(idx_hbm.at[pl.ds(base, chunk)], idx_v.at[buf])
            nxt = pltpu.async_copy(
                p_hbm.at[idx_v.at[buf]], rows_v.at[buf], sems[buf]
            )
            pending.wait()
            pltpu.sync_copy(
                rows_v.at[1 - buf],
                out_hbm.at[pl.ds(base - chunk, chunk)],
            )
            pending = nxt
        pending.wait()
        last = n_chunks - 1
        pltpu.sync_copy(
            rows_v.at[last % 2],
            out_hbm.at[pl.ds(wid * b_per_w + last * chunk, chunk)],
        )

    return gather


def _finish_body(in_ref, out_ref):
    x3 = in_ref[...].reshape(8, 512, 128)
    for s in range(8):
        xt = x3[s].T
        for j in range(10):
            out_ref[j, s, :] = jnp.concatenate(
                [xt[16 * k + j] for k in range(8)], axis=0
            )


def _finish(out16_flat, S, B):
    n_tok = S * B
    view = out16_flat.reshape(n_tok // 8, 128)
    return pl.pallas_call(
        _finish_body,
        grid=(S // 8,),
        in_specs=[pl.BlockSpec((8 * B * _PADDED_OUT // 128, 128),
                               lambda i: (i, 0))],
        out_specs=pl.BlockSpec((10, 8, B), lambda i: (0, i, 0)),
        out_shape=jax.ShapeDtypeStruct((10, S, B), jnp.float32),
    )(view)


def kernel(input, table, W1, b1, W2, b2):
    B, S = input.shape
    n_tok = B * S

    W2p = jnp.pad(W2, ((0, 0), (0, _PADDED_OUT - W2.shape[1])))
    b2p = jnp.pad(b2, (0, _PADDED_OUT - b2.shape[0]))

    packed = _project_table(
        table.T, W1, b1.reshape(1, -1), W2p, b2p.reshape(1, -1)
    )
    proj = packed.reshape(_VOCAB_PAD * _PADDED_OUT).reshape(
        _VOCAB_PAD, _PADDED_OUT
    )

    idx_flat = (
        input.T.reshape(S, B // 512, 512)
        .swapaxes(1, 2)
        .reshape(n_tok)
        .astype(jnp.int32)
    )
    idx_rows = ((idx_flat & (_N_PACKED - 1)) << 3) | (
        lax.shift_right_logical(idx_flat, 17)
    )
    out16 = _make_gather(n_tok, 2560)(proj, idx_rows)
    otp = _finish(out16.reshape(n_tok * _PADDED_OUT), S, B)
    return jnp.transpose(otp, (2, 1, 0))
